# baseline (speedup 1.0000x reference)
import functools

import jax
import jax.numpy as jnp
from jax import lax
from jax.experimental import pallas as pl
from jax.experimental.pallas import tpu as pltpu
from jax.experimental.pallas import tpu_sc as plsc

N, D, B, T = 100000, 128, 4096, 8
H1, H2 = 128, 64
S1, S2 = 5, 2
W_POS, W_NOPOS = 0.6, 0.4

NW = 32
CH = 128
DW = D // 2
G1_ROWS = S1 * T * B
A1_ROWS = S1 * T * B
H2_ROWS = S2 * A1_ROWS
G1_PW = G1_ROWS // NW
H2_PW = H2_ROWS // NW
H0_PW = B // NW


def _sc_gather(x, idx1, idx2, nodes):
    mesh = plsc.VectorSubcoreMesh(core_axis_name="c", subcore_axis_name="s")

    @functools.partial(
        pl.kernel,
        out_type=(
            jax.ShapeDtypeStruct((G1_ROWS, DW), jnp.float32),
            jax.ShapeDtypeStruct((H2_ROWS, DW), jnp.float32),
            jax.ShapeDtypeStruct((B, DW), jnp.float32),
        ),
        mesh=mesh,
        scratch_types=[
            pltpu.VMEM((H2_PW,), jnp.int32),
            pltpu.VMEM((CH, DW), jnp.float32),
            pltpu.VMEM((CH, DW), jnp.float32),
            pltpu.SemaphoreType.DMA,
            pltpu.SemaphoreType.DMA,
        ],
        compiler_params=pltpu.CompilerParams(use_tc_tiling_on_sc=False),
    )
    def k(x_hbm, idx1_hbm, idx2_hbm, nodes_hbm, g1_hbm, h2_hbm, h0_hbm,
          idx_v, buf_a, buf_b, sem_a, sem_b):
        wid = lax.axis_index("s") * 2 + lax.axis_index("c")

        def start_gather(c, buf, sem):
            pltpu.async_copy(x_hbm.at[idx_v.at[pl.ds(c * CH, CH)]], buf, sem)

        def wait_gather(c, buf, sem):
            pltpu.make_async_copy(
                x_hbm.at[idx_v.at[pl.ds(c * CH, CH)]], buf, sem).wait()

        def copy_phase(idx_hbm, n_pw, out_hbm):
            base = wid * n_pw
            pltpu.sync_copy(idx_hbm.at[pl.ds(base, n_pw)],
                            idx_v.at[pl.ds(0, n_pw)])
            nch = n_pw // CH
            start_gather(0, buf_a, sem_a)

            def body(g, _):
                c0 = 2 * g

                @pl.when(c0 + 1 < nch)
                def _():
                    start_gather(c0 + 1, buf_b, sem_b)

                wait_gather(c0, buf_a, sem_a)
                pltpu.sync_copy(buf_a, out_hbm.at[pl.ds(base + c0 * CH, CH)])

                @pl.when(c0 + 1 < nch)
                def _():

                    @pl.when(c0 + 2 < nch)
                    def _():
                        start_gather(c0 + 2, buf_a, sem_a)

                    wait_gather(c0 + 1, buf_b, sem_b)
                    pltpu.sync_copy(
                        buf_b, out_hbm.at[pl.ds(base + (c0 + 1) * CH, CH)])

                return 0

            lax.fori_loop(0, (nch + 1) // 2, body, 0)

        copy_phase(idx1_hbm, G1_PW, g1_hbm)
        copy_phase(idx2_hbm, H2_PW, h2_hbm)
        copy_phase(nodes_hbm, H0_PW, h0_hbm)

    return k(x, idx1, idx2, nodes)


def _tc_dense(g1, h2, h0, w1s, w1n, w2s, w2n, awp, awn, pe, wout, bout):
    NB = 16
    BT = B // NB

    def body(g1r, h2r, h0r, w1sr, w1nr, w2sr, w2nr, awpr, awnr, per,
             woutr, boutr, outr, seq):
        t = pl.program_id(1)
        w1s_ = w1sr[...]
        w1n_ = w1nr[...]
        g = [g1r[s, 0].astype(jnp.float32) for s in range(S1)]
        agg0 = (g[0] + g[1] + g[2] + g[3] + g[4]) * (1.0 / S1)
        h0f = h0r[...].astype(jnp.float32)
        z0 = jnp.maximum(h0f @ w1s_ + agg0 @ w1n_, 0.0)
        zsum = None
        for s in range(S1):
            a1f = (h2r[s, 0].astype(jnp.float32)
                   + h2r[s + S1, 0].astype(jnp.float32)) * 0.5
            z1 = jnp.maximum(g[s] @ w1s_ + a1f @ w1n_, 0.0)
            zsum = z1 if zsum is None else zsum + z1
        agg2 = zsum * (1.0 / S1)
        z2 = jnp.maximum(z0 @ w2sr[...] + agg2 @ w2nr[...], 0.0)
        seq[pl.ds(t, 1)] = z2[None]

        @pl.when(t == T - 1)
        def _():
            sq = seq[...]

            def attn(s_, w_):
                sc_ = jnp.sum(s_ * w_[None, None, :], axis=-1, keepdims=True)
                m = jnp.max(sc_, axis=0, keepdims=True)
                e = jnp.exp(sc_ - m)
                wt = e / jnp.sum(e, axis=0, keepdims=True)
                return jnp.sum(s_ * wt, axis=0)

            pe_ = per[...]
            awp_ = awpr[...]
            awn_ = awnr[...]
            emb0 = (attn(sq + pe_[:, None, :], awp_[0]) * W_POS
                    + attn(sq, awn_[0]) * W_NOPOS)
            sq1 = jnp.stack([sq[0], sq[2], sq[4], sq[6]])
            emb1 = (attn(sq1 + pe_[0:4][:, None, :], awp_[1]) * W_POS
                    + attn(sq1, awn_[1]) * W_NOPOS)
            stacked = (emb0 + emb1) * 0.5
            outr[...] = stacked @ woutr[...] + boutr[...]

    return pl.pallas_call(
        body,
        grid=(NB, T),
        in_specs=[
            pl.BlockSpec((S1, 1, BT, D), lambda b, t: (0, t, b, 0)),
            pl.BlockSpec((S2 * S1, 1, BT, D), lambda b, t: (0, t, b, 0)),
            pl.BlockSpec((BT, D), lambda b, t: (b, 0)),
            pl.BlockSpec((D, H1), lambda b, t: (0, 0)),
            pl.BlockSpec((D, H1), lambda b, t: (0, 0)),
            pl.BlockSpec((H1, H2), lambda b, t: (0, 0)),
            pl.BlockSpec((H1, H2), lambda b, t: (0, 0)),
            pl.BlockSpec((2, H2), lambda b, t: (0, 0)),
            pl.BlockSpec((2, H2), lambda b, t: (0, 0)),
            pl.BlockSpec((T, H2), lambda b, t: (0, 0)),
            pl.BlockSpec((H2, D), lambda b, t: (0, 0)),
            pl.BlockSpec((1, D), lambda b, t: (0, 0)),
        ],
        out_specs=pl.BlockSpec((BT, D), lambda b, t: (b, 0)),
        out_shape=jax.ShapeDtypeStruct((B, D), jnp.float32),
        scratch_shapes=[pltpu.VMEM((T, BT, H2), jnp.float32)],
        compiler_params=pltpu.CompilerParams(
            dimension_semantics=("arbitrary", "arbitrary"),
        ),
    )(g1, h2, h0, w1s, w1n, w2s, w2n, awp, awn, pe, wout, bout)


def kernel(x, nodes, nbr1, nbr2, W1_self, W1_nbr, W2_self, W2_nbr,
           attn_w_pos, attn_b_pos, attn_w_nopos, attn_b_nopos, pe, Wout,
           bout):
    del attn_b_pos, attn_b_nopos
    x_v = lax.bitcast_convert_type(
        x.astype(jnp.bfloat16).reshape(N, DW, 2), jnp.float32)
    nodes_i = nodes.astype(jnp.int32).reshape(B)
    idx1 = jnp.transpose(nbr1.astype(jnp.int32).reshape(T, B, S1),
                         (2, 0, 1)).reshape(G1_ROWS)
    idx2 = jnp.transpose(nbr2.astype(jnp.int32).reshape(T, B, S1, S2),
                         (3, 2, 0, 1)).reshape(H2_ROWS)
    g1, h2, h0 = _sc_gather(x_v, idx1, idx2, nodes_i)

    def as_bf(v, *lead):
        return lax.bitcast_convert_type(v, jnp.bfloat16).reshape(*lead, D)

    return _tc_dense(
        as_bf(g1, S1, T, B), as_bf(h2, S2 * S1, T, B), as_bf(h0, B),
        W1_self, W1_nbr, W2_self, W2_nbr,
        attn_w_pos, attn_w_nopos, pe, Wout, bout.reshape(1, D))


# speedup vs baseline: 5.6614x; 5.6614x over previous
import functools

import jax
import jax.numpy as jnp
from jax import lax
from jax.experimental import pallas as pl
from jax.experimental.pallas import tpu as pltpu
from jax.experimental.pallas import tpu_sc as plsc

N, D, B, T = 100000, 128, 4096, 8
H1, H2 = 128, 64
S1, S2 = 5, 2
W_POS, W_NOPOS = 0.6, 0.4

NW = 32
CH = 128
G1_ROWS = S1 * T * B
A1_ROWS = S1 * T * B
H2_ROWS = S2 * A1_ROWS
G1_PW = G1_ROWS // NW
A1_PW = A1_ROWS // NW
H0_PW = B // NW


def _sc_gather(x, idx1, idx2, nodes):
    mesh = plsc.VectorSubcoreMesh(core_axis_name="c", subcore_axis_name="s")

    @functools.partial(
        pl.kernel,
        out_type=(
            jax.ShapeDtypeStruct((G1_ROWS, D), jnp.float32),
            jax.ShapeDtypeStruct((A1_ROWS, D), jnp.float32),
            jax.ShapeDtypeStruct((B, D), jnp.float32),
        ),
        mesh=mesh,
        scratch_types=[
            pltpu.VMEM((2 * A1_PW,), jnp.int32),
            pltpu.VMEM((CH, D), jnp.float32),
            pltpu.VMEM((CH, D), jnp.float32),
            pltpu.VMEM((CH, D), jnp.float32),
            pltpu.VMEM((CH, D), jnp.float32),
            pltpu.VMEM((CH, D), jnp.float32),
            pltpu.SemaphoreType.DMA,
            pltpu.SemaphoreType.DMA,
            pltpu.SemaphoreType.DMA,
            pltpu.SemaphoreType.DMA,
        ],
    )
    def k(x_hbm, idx1_hbm, idx2_hbm, nodes_hbm, g1_hbm, a1_hbm, h0_hbm,
          idx_v, buf_a, buf_b, buf_c, buf_d, obuf, sem_a, sem_b, sem_c,
          sem_d):
        wid = lax.axis_index("s") * 2 + lax.axis_index("c")

        def start_gather(c, buf, sem):
            pltpu.async_copy(x_hbm.at[idx_v.at[pl.ds(c * CH, CH)]], buf, sem)

        def wait_gather(c, buf, sem):
            pltpu.make_async_copy(
                x_hbm.at[idx_v.at[pl.ds(c * CH, CH)]], buf, sem).wait()

        def copy_phase(idx_hbm, n_pw, out_hbm):
            base = wid * n_pw
            pltpu.sync_copy(idx_hbm.at[pl.ds(base, n_pw)],
                            idx_v.at[pl.ds(0, n_pw)])
            nch = n_pw // CH
            start_gather(0, buf_a, sem_a)

            def body(g, _):
                c0 = 2 * g

                @pl.when(c0 + 1 < nch)
                def _():
                    start_gather(c0 + 1, buf_b, sem_b)

                wait_gather(c0, buf_a, sem_a)
                pltpu.sync_copy(buf_a, out_hbm.at[pl.ds(base + c0 * CH, CH)])

                @pl.when(c0 + 1 < nch)
                def _():

                    @pl.when(c0 + 2 < nch)
                    def _():
                        start_gather(c0 + 2, buf_a, sem_a)

                    wait_gather(c0 + 1, buf_b, sem_b)
                    pltpu.sync_copy(
                        buf_b, out_hbm.at[pl.ds(base + (c0 + 1) * CH, CH)])

                return 0

            lax.fori_loop(0, (nch + 1) // 2, body, 0)

        def reduce_phase():
            base = wid * A1_PW
            pltpu.sync_copy(idx2_hbm.at[pl.ds(base, A1_PW)],
                            idx_v.at[pl.ds(0, A1_PW)])
            pltpu.sync_copy(idx2_hbm.at[pl.ds(A1_ROWS + base, A1_PW)],
                            idx_v.at[pl.ds(A1_PW, A1_PW)])
            nch = A1_PW // CH

            def startpair(c, b0, b1, s0, s1):
                pltpu.async_copy(
                    x_hbm.at[idx_v.at[pl.ds(c * CH, CH)]], b0, s0)
                pltpu.async_copy(
                    x_hbm.at[idx_v.at[pl.ds(A1_PW + c * CH, CH)]], b1, s1)

            def waitpair(c, b0, b1, s0, s1):
                pltpu.make_async_copy(
                    x_hbm.at[idx_v.at[pl.ds(c * CH, CH)]], b0, s0).wait()
                pltpu.make_async_copy(
                    x_hbm.at[idx_v.at[pl.ds(A1_PW + c * CH, CH)]], b1,
                    s1).wait()

            def pair_add(b0, b1):
                def body(r, _):
                    for j in range(D // 16):
                        sl = pl.ds(j * 16, 16)
                        obuf[r, sl] = b0[r, sl] + b1[r, sl]
                    return 0

                lax.fori_loop(0, CH, body, 0)

            startpair(0, buf_a, buf_b, sem_a, sem_b)

            def body(g, _):
                c0 = 2 * g
                startpair(c0 + 1, buf_c, buf_d, sem_c, sem_d)
                waitpair(c0, buf_a, buf_b, sem_a, sem_b)
                pair_add(buf_a, buf_b)
                pltpu.sync_copy(obuf, a1_hbm.at[pl.ds(base + c0 * CH, CH)])

                @pl.when(c0 + 2 < nch)
                def _():
                    startpair(c0 + 2, buf_a, buf_b, sem_a, sem_b)

                waitpair(c0 + 1, buf_c, buf_d, sem_c, sem_d)
                pair_add(buf_c, buf_d)
                pltpu.sync_copy(obuf,
                                a1_hbm.at[pl.ds(base + (c0 + 1) * CH, CH)])
                return 0

            lax.fori_loop(0, nch // 2, body, 0)

        copy_phase(idx1_hbm, G1_PW, g1_hbm)
        reduce_phase()
        copy_phase(nodes_hbm, H0_PW, h0_hbm)

    return k(x, idx1, idx2, nodes)


def _tc_dense(g1, a1, h0, w1s, w1n, w2s, w2n, awp, awn, pe, wout, bout):
    NB = 16
    BT = B // NB

    def body(g1r, a1r, h0r, w1sr, w1nr, w2sr, w2nr, awpr, awnr, per,
             woutr, boutr, outr, seq):
        t = pl.program_id(1)
        w1s_ = w1sr[...]
        w1n_ = w1nr[...]
        g = [g1r[s, 0] for s in range(S1)]
        agg0 = (g[0] + g[1] + g[2] + g[3] + g[4]) * (1.0 / S1)
        z0 = jnp.maximum(h0r[...] @ w1s_ + agg0 @ w1n_, 0.0)
        zsum = None
        for s in range(S1):
            a1f = a1r[s, 0] * 0.5
            z1 = jnp.maximum(g[s] @ w1s_ + a1f @ w1n_, 0.0)
            zsum = z1 if zsum is None else zsum + z1
        agg2 = zsum * (1.0 / S1)
        z2 = jnp.maximum(z0 @ w2sr[...] + agg2 @ w2nr[...], 0.0)
        seq[pl.ds(t, 1)] = z2[None]

        @pl.when(t == T - 1)
        def _():
            sq = seq[...]

            def attn(s_, w_):
                sc_ = jnp.sum(s_ * w_[None, None, :], axis=-1, keepdims=True)
                m = jnp.max(sc_, axis=0, keepdims=True)
                e = jnp.exp(sc_ - m)
                wt = e / jnp.sum(e, axis=0, keepdims=True)
                return jnp.sum(s_ * wt, axis=0)

            pe_ = per[...]
            awp_ = awpr[...]
            awn_ = awnr[...]
            emb0 = (attn(sq + pe_[:, None, :], awp_[0]) * W_POS
                    + attn(sq, awn_[0]) * W_NOPOS)
            sq1 = jnp.stack([sq[0], sq[2], sq[4], sq[6]])
            emb1 = (attn(sq1 + pe_[0:4][:, None, :], awp_[1]) * W_POS
                    + attn(sq1, awn_[1]) * W_NOPOS)
            stacked = (emb0 + emb1) * 0.5
            outr[...] = stacked @ woutr[...] + boutr[...]

    return pl.pallas_call(
        body,
        grid=(NB, T),
        in_specs=[
            pl.BlockSpec((S1, 1, BT, D), lambda b, t: (0, t, b, 0)),
            pl.BlockSpec((S1, 1, BT, D), lambda b, t: (0, t, b, 0)),
            pl.BlockSpec((BT, D), lambda b, t: (b, 0)),
            pl.BlockSpec((D, H1), lambda b, t: (0, 0)),
            pl.BlockSpec((D, H1), lambda b, t: (0, 0)),
            pl.BlockSpec((H1, H2), lambda b, t: (0, 0)),
            pl.BlockSpec((H1, H2), lambda b, t: (0, 0)),
            pl.BlockSpec((2, H2), lambda b, t: (0, 0)),
            pl.BlockSpec((2, H2), lambda b, t: (0, 0)),
            pl.BlockSpec((T, H2), lambda b, t: (0, 0)),
            pl.BlockSpec((H2, D), lambda b, t: (0, 0)),
            pl.BlockSpec((1, D), lambda b, t: (0, 0)),
        ],
        out_specs=pl.BlockSpec((BT, D), lambda b, t: (b, 0)),
        out_shape=jax.ShapeDtypeStruct((B, D), jnp.float32),
        scratch_shapes=[pltpu.VMEM((T, BT, H2), jnp.float32)],
        compiler_params=pltpu.CompilerParams(
            dimension_semantics=("arbitrary", "arbitrary"),
        ),
    )(g1, a1, h0, w1s, w1n, w2s, w2n, awp, awn, pe, wout, bout)


def kernel(x, nodes, nbr1, nbr2, W1_self, W1_nbr, W2_self, W2_nbr,
           attn_w_pos, attn_b_pos, attn_w_nopos, attn_b_nopos, pe, Wout,
           bout):
    del attn_b_pos, attn_b_nopos
    nodes_i = nodes.astype(jnp.int32).reshape(B)
    idx1 = jnp.transpose(nbr1.astype(jnp.int32).reshape(T, B, S1),
                         (2, 0, 1)).reshape(G1_ROWS)
    idx2 = jnp.transpose(nbr2.astype(jnp.int32).reshape(T, B, S1, S2),
                         (3, 2, 0, 1)).reshape(H2_ROWS)
    g1, a1, h0 = _sc_gather(x, idx1, idx2, nodes_i)
    return _tc_dense(
        g1.reshape(S1, T, B, D), a1.reshape(S1, T, B, D), h0,
        W1_self, W1_nbr, W2_self, W2_nbr,
        attn_w_pos, attn_w_nopos, pe, Wout, bout.reshape(1, D))
